# baseline (speedup 1.0000x reference)
import functools

import jax
import jax.numpy as jnp
from jax import lax
from jax.experimental import pallas as pl
from jax.experimental.pallas import tpu as pltpu
from jax.experimental.pallas import tpu_sc as plsc

N_NODES = 50000
D_FEAT = 128
LANES = 16
N_SUBCORES = 16
N_CORES = 2

EB = 128
NB = 80
ET = NB * EB
E_PAD = N_SUBCORES * ET
C_ROWS = 6272
N_CHUNKS = 8
N_PAD = N_CHUNKS * C_ROWS
ACC_ROWS = C_ROWS + 8
DEG_PAD = 16 * 3136
ROWS_PER_TILE = C_ROWS // N_SUBCORES

ROW_BLK = 1000


def _rsqrt_sc(z):
    yi = jnp.int32(0x5F3759DF) - (lax.bitcast_convert_type(z, jnp.int32) >> 1)
    y = lax.bitcast_convert_type(yi, jnp.float32)
    for _ in range(3):
        y = y * (1.5 - 0.5 * z * y * y)
    return y


def _sc_agg_body(x_hbm, src_hbm, dst_hbm, val_hbm, out_hbm,
                 src_v, dst_v, val_v, rows, off_r, wm_r, dsb, ddb,
                 zb1, zb2, deg_s, deg_d, acc, p1sem, p2sem, gsem, ssem):
    t = lax.axis_index("s")
    core = lax.axis_index("c")

    pltpu.sync_copy(src_hbm.at[pl.ds(t * NB, NB)], src_v)
    pltpu.sync_copy(dst_hbm.at[pl.ds(t * NB, NB)], dst_v)
    pltpu.sync_copy(val_hbm.at[pl.ds(t * NB, NB)], val_v)

    zero16 = jnp.zeros((LANES,), jnp.float32)

    def _z1(i, _):
        zb1[pl.ds(i * LANES, LANES)] = zero16
        return 0
    lax.fori_loop(0, 3136 // LANES, _z1, 0)

    def _z2(i, _):
        for k in range(8):
            zb2[i, pl.ds(k * LANES, LANES)] = zero16
        return 0
    lax.fori_loop(0, 8, _z2, 0)

    pltpu.sync_copy(zb1, deg_s.at[pl.ds(t * 3136, 3136)])
    pltpu.sync_copy(zb1, deg_d.at[pl.ds(t * 3136, 3136)])
    plsc.subcore_barrier()

    for b in range(3):
        pltpu.async_copy(val_v.at[b], deg_s.at[src_v.at[b]], p1sem, add=True)
        pltpu.async_copy(val_v.at[b], deg_d.at[dst_v.at[b]], p1sem, add=True)

    def _p1(b, _):
        @pl.when(b <= NB - 4)
        def _():
            pltpu.async_copy(val_v.at[b + 3], deg_s.at[src_v.at[b + 3]],
                             p1sem, add=True)
            pltpu.async_copy(val_v.at[b + 3], deg_d.at[dst_v.at[b + 3]],
                             p1sem, add=True)
        pltpu.make_async_copy(val_v.at[b], deg_s.at[src_v.at[b]], p1sem).wait()
        pltpu.make_async_copy(val_v.at[b], deg_d.at[dst_v.at[b]], p1sem).wait()
        return 0
    lax.fori_loop(0, NB, _p1, 0)
    plsc.subcore_barrier()

    for b in range(6):
        pltpu.async_copy(deg_s.at[src_v.at[b]], dsb.at[b % 8], p2sem)
        pltpu.async_copy(deg_d.at[dst_v.at[b]], ddb.at[b % 8], p2sem)

    def _p2(b, _):
        @pl.when(b <= NB - 7)
        def _():
            s = lax.rem(b + 6, 8)
            pltpu.async_copy(deg_s.at[src_v.at[b + 6]], dsb.at[s], p2sem)
            pltpu.async_copy(deg_d.at[dst_v.at[b + 6]], ddb.at[s], p2sem)
        s = lax.rem(b, 8)
        pltpu.make_async_copy(deg_s.at[src_v.at[b]], dsb.at[s], p2sem).wait()
        pltpu.make_async_copy(deg_d.at[dst_v.at[b]], ddb.at[s], p2sem).wait()
        for g in range(8):
            sl = pl.ds(g * LANES, LANES)
            ds16 = dsb[s, sl]
            dd16 = ddb[s, sl]
            va16 = val_v[b, sl]
            z = jnp.maximum(ds16, 1e-12) * jnp.maximum(dd16, 1e-12)
            val_v[b, sl] = va16 * _rsqrt_sc(z)
        return 0
    lax.fori_loop(0, NB, _p2, 0)

    def _pass(p, _):
        cbase = (core * (N_CHUNKS // 2) + p) * C_ROWS

        def _za(i, _):
            pltpu.sync_copy(zb2, acc.at[pl.ds(t * ROWS_PER_TILE + i * 8, 8)])
            return 0
        lax.fori_loop(0, ROWS_PER_TILE // 8, _za, 0)
        plsc.subcore_barrier()

        pltpu.async_copy(x_hbm.at[src_v.at[0]], rows.at[0], gsem)

        def _edge(b, _):
            slot = lax.rem(b, 2)
            pltpu.make_async_copy(x_hbm.at[src_v.at[b]], rows.at[slot],
                                  gsem).wait()
            for g in range(8):
                sl = pl.ds(g * LANES, LANES)
                d16 = dst_v[b, sl]
                off16 = d16 - cbase
                valid = (off16 >= 0) & (off16 < C_ROWS)
                off_r[slot, sl] = jnp.where(valid, off16, C_ROWS)
                wm_r[slot, sl] = jnp.where(valid, val_v[b, sl], 0.0)

            def _scale(g, _):
                wm16 = wm_r[slot, pl.ds(g * LANES, LANES)]
                for j2 in range(LANES):
                    e = g * LANES + j2
                    bc = jnp.full((LANES,), wm16[j2])
                    for k in range(8):
                        slk = pl.ds(k * LANES, LANES)
                        rows[slot, e, slk] = rows[slot, e, slk] * bc
                return 0
            lax.fori_loop(0, 8, _scale, 0)

            pltpu.async_copy(rows.at[slot], acc.at[off_r.at[slot]], ssem,
                             add=True)

            @pl.when(b >= 1)
            def _():
                pltpu.make_async_copy(rows.at[0], acc.at[off_r.at[0]],
                                      ssem).wait()

            @pl.when(b <= NB - 2)
            def _():
                pltpu.async_copy(x_hbm.at[src_v.at[b + 1]],
                                 rows.at[lax.rem(b + 1, 2)], gsem)
            return 0
        lax.fori_loop(0, NB, _edge, 0)

        pltpu.make_async_copy(rows.at[0], acc.at[off_r.at[0]], ssem).wait()
        plsc.subcore_barrier()

        pltpu.sync_copy(
            acc.at[pl.ds(t * ROWS_PER_TILE, ROWS_PER_TILE)],
            out_hbm.at[pl.ds(cbase + t * ROWS_PER_TILE, ROWS_PER_TILE)])
        return 0
    lax.fori_loop(0, N_CHUNKS // 2, _pass, 0)


def _sc_agg(x, src_r, dst_r, val_r):
    mesh = plsc.VectorSubcoreMesh(core_axis_name="c", subcore_axis_name="s",
                                  num_cores=N_CORES, num_subcores=N_SUBCORES)
    f = pl.kernel(
        _sc_agg_body,
        out_type=jax.ShapeDtypeStruct((N_PAD, D_FEAT), jnp.float32),
        mesh=mesh,
        scratch_types=[
            pltpu.VMEM((NB, EB), jnp.int32),
            pltpu.VMEM((NB, EB), jnp.int32),
            pltpu.VMEM((NB, EB), jnp.float32),
            pltpu.VMEM((2, EB, D_FEAT), jnp.float32),
            pltpu.VMEM((2, EB), jnp.int32),
            pltpu.VMEM((2, EB), jnp.float32),
            pltpu.VMEM((8, EB), jnp.float32),
            pltpu.VMEM((8, EB), jnp.float32),
            pltpu.VMEM((3136,), jnp.float32),
            pltpu.VMEM((8, D_FEAT), jnp.float32),
            pltpu.VMEM_SHARED((DEG_PAD,), jnp.float32),
            pltpu.VMEM_SHARED((DEG_PAD,), jnp.float32),
            pltpu.VMEM_SHARED((ACC_ROWS, D_FEAT), jnp.float32),
            pltpu.SemaphoreType.DMA,
            pltpu.SemaphoreType.DMA,
            pltpu.SemaphoreType.DMA,
            pltpu.SemaphoreType.DMA,
        ],
    )
    return f(x, src_r, dst_r, val_r)


def _prep_edges(edge_index, val):
    e = val.shape[0]
    pad = E_PAD - e
    src = jnp.pad(edge_index[0], (0, pad)).reshape(E_PAD // EB, EB)
    dst = jnp.pad(edge_index[1], (0, pad)).reshape(E_PAD // EB, EB)
    v = jnp.pad(val, (0, pad)).reshape(E_PAD // EB, EB)
    return src, dst, v


def _dense_stage_body(a1_ref, a2_ref, w1_ref, w2_ref, b1_ref, b2_ref, o_ref):
    h1 = jnp.dot(a1_ref[...], w1_ref[...], preferred_element_type=jnp.float32)
    h2 = jnp.dot(a2_ref[...], w2_ref[...], preferred_element_type=jnp.float32)
    r1 = jnp.maximum(h1 + b1_ref[...], 0.0)
    r2 = jnp.maximum(h2 + b2_ref[...], 0.0)
    o_ref[...] = (r1 + r2) * 0.5


def _dense_stage(a1, a2, W1, W2, b1, b2):
    grid = (N_NODES // ROW_BLK,)
    blk = lambda i: (i, 0)
    full = lambda i: (0, 0)
    return pl.pallas_call(
        _dense_stage_body,
        grid=grid,
        in_specs=[
            pl.BlockSpec((ROW_BLK, D_FEAT), blk),
            pl.BlockSpec((ROW_BLK, D_FEAT), blk),
            pl.BlockSpec((D_FEAT, D_FEAT), full),
            pl.BlockSpec((D_FEAT, D_FEAT), full),
            pl.BlockSpec((1, D_FEAT), full),
            pl.BlockSpec((1, D_FEAT), full),
        ],
        out_specs=pl.BlockSpec((ROW_BLK, D_FEAT), blk),
        out_shape=jax.ShapeDtypeStruct((N_NODES, D_FEAT), jnp.float32),
    )(a1, a2, W1, W2, b1.reshape(1, -1), b2.reshape(1, -1))


def kernel(com_emb, pos_emb, demand_edge_index, supply_edge_index,
           comflow_edge_index, posflow_edge_index,
           demand_val, supply_val, comflow_val, posflow_val,
           W_demand, b_demand, W_supply, b_supply,
           W_comflow, b_comflow, W_posflow, b_posflow):
    d_src, d_dst, d_val = _prep_edges(demand_edge_index, demand_val)
    s_src, s_dst, s_val = _prep_edges(supply_edge_index, supply_val)
    c_src, c_dst, c_val = _prep_edges(comflow_edge_index, comflow_val)
    p_src, p_dst, p_val = _prep_edges(posflow_edge_index, posflow_val)

    agg_d = _sc_agg(com_emb, d_src, d_dst, d_val)
    agg_s = _sc_agg(pos_emb, s_src, s_dst, s_val)
    agg_c = _sc_agg(com_emb, c_src, c_dst, c_val)
    agg_p = _sc_agg(pos_emb, p_src, p_dst, p_val)

    com_out = _dense_stage(agg_s, agg_c, W_supply, W_comflow, b_supply, b_comflow)
    pos_out = _dense_stage(agg_d, agg_p, W_demand, W_posflow, b_demand, b_posflow)
    return (com_out, pos_out)


# speedup vs baseline: 1.0824x; 1.0824x over previous
import functools

import jax
import jax.numpy as jnp
from jax import lax
from jax.experimental import pallas as pl
from jax.experimental.pallas import tpu as pltpu
from jax.experimental.pallas import tpu_sc as plsc

N_NODES = 50000
D_FEAT = 128
LANES = 16
N_SUBCORES = 16
N_CORES = 2

EB = 128
SB = 64
NB = 80
NSB = NB * 2
ET = NB * EB
E_PAD = N_SUBCORES * ET
C_ROWS = 6272
N_CHUNKS = 8
N_PAD = N_CHUNKS * C_ROWS
ACC_ROWS = C_ROWS + 8
ROWS_PER_TILE = C_ROWS // N_SUBCORES

ROW_BLK = 1000


def _rsqrt_sc(z):
    yi = jnp.int32(0x5F3759DF) - (lax.bitcast_convert_type(z, jnp.int32) >> 1)
    y = lax.bitcast_convert_type(yi, jnp.float32)
    for _ in range(3):
        y = y * (1.5 - 0.5 * z * y * y)
    return y


def _sc_agg_body(x_hbm, src_hbm, dst_hbm, val_hbm, out_hbm,
                 src_v, dst_v, val_v, rows, off_r, wm_r, dsb, ddb,
                 zb1, zb2, deg_s, deg_d, acc, p1sem, p2sem, gsem, ssem):
    t = lax.axis_index("s")
    core = lax.axis_index("c")

    pltpu.sync_copy(src_hbm.at[pl.ds(t * NB, NB)], src_v)
    pltpu.sync_copy(dst_hbm.at[pl.ds(t * NB, NB)], dst_v)
    pltpu.sync_copy(val_hbm.at[pl.ds(t * NB, NB)], val_v)

    zero16 = jnp.zeros((LANES,), jnp.float32)

    def _z1(i, _):
        zb1[pl.ds(i * LANES, LANES)] = zero16
        return 0
    lax.fori_loop(0, 3136 // LANES, _z1, 0)

    def _z2(i, _):
        for k in range(8):
            zb2[i, pl.ds(k * LANES, LANES)] = zero16
        return 0
    lax.fori_loop(0, 8, _z2, 0)

    pltpu.sync_copy(zb1, deg_s.at[pl.ds(t * 3136, 3136)])
    pltpu.sync_copy(zb1, deg_d.at[pl.ds(t * 3136, 3136)])
    plsc.subcore_barrier()

    W1 = 8
    for b in range(W1):
        pltpu.async_copy(val_v.at[b], deg_s.at[src_v.at[b]], p1sem, add=True)
        pltpu.async_copy(val_v.at[b], deg_d.at[dst_v.at[b]], p1sem, add=True)

    def _p1(b, _):
        @pl.when(b <= NB - W1 - 1)
        def _():
            pltpu.async_copy(val_v.at[b + W1], deg_s.at[src_v.at[b + W1]],
                             p1sem, add=True)
            pltpu.async_copy(val_v.at[b + W1], deg_d.at[dst_v.at[b + W1]],
                             p1sem, add=True)
        pltpu.make_async_copy(val_v.at[b], deg_s.at[src_v.at[b]], p1sem).wait()
        pltpu.make_async_copy(val_v.at[b], deg_d.at[dst_v.at[b]], p1sem).wait()
        return 0
    lax.fori_loop(0, NB, _p1, 0)
    plsc.subcore_barrier()

    W2 = 6
    for b in range(W2):
        pltpu.async_copy(deg_s.at[src_v.at[b]], dsb.at[b % 8], p2sem)
        pltpu.async_copy(deg_d.at[dst_v.at[b]], ddb.at[b % 8], p2sem)

    def _p2(b, _):
        @pl.when(b <= NB - W2 - 1)
        def _():
            s = lax.rem(b + W2, 8)
            pltpu.async_copy(deg_s.at[src_v.at[b + W2]], dsb.at[s], p2sem)
            pltpu.async_copy(deg_d.at[dst_v.at[b + W2]], ddb.at[s], p2sem)
        s = lax.rem(b, 8)
        pltpu.make_async_copy(deg_s.at[src_v.at[b]], dsb.at[s], p2sem).wait()
        pltpu.make_async_copy(deg_d.at[dst_v.at[b]], ddb.at[s], p2sem).wait()
        dsr = dsb.at[s]
        ddr = ddb.at[s]
        vvr = val_v.at[b]
        for g in range(EB // LANES):
            sl = pl.ds(g * LANES, LANES)
            z = (jnp.maximum(dsr[sl], 1e-12) * jnp.maximum(ddr[sl], 1e-12))
            vvr[sl] = vvr[sl] * _rsqrt_sc(z)
        return 0
    lax.fori_loop(0, NB, _p2, 0)

    def _sub(b):
        return lax.div(b, 2), lax.rem(b, 2) * SB

    def _pass(p, _):
        cbase = (core * (N_CHUNKS // 2) + p) * C_ROWS

        def _za(i, _):
            pltpu.sync_copy(zb2, acc.at[pl.ds(t * ROWS_PER_TILE + i * 8, 8)])
            return 0
        lax.fori_loop(0, ROWS_PER_TILE // 8, _za, 0)
        plsc.subcore_barrier()

        pltpu.async_copy(x_hbm.at[src_v.at[0, pl.ds(0, SB)]], rows.at[0], gsem)
        pltpu.async_copy(x_hbm.at[src_v.at[0, pl.ds(SB, SB)]], rows.at[1],
                         gsem)

        def _edge(b, _):
            slot = lax.rem(b, 4)
            row, hh = _sub(b)
            pltpu.make_async_copy(x_hbm.at[src_v.at[row, pl.ds(hh, SB)]],
                                  rows.at[slot], gsem).wait()
            rr = rows.at[slot]
            orr = off_r.at[slot]
            wrr = wm_r.at[slot]
            dvr = dst_v.at[row]
            vvr = val_v.at[row]
            for g in range(SB // LANES):
                sl = pl.ds(hh + g * LANES, LANES)
                slo = pl.ds(g * LANES, LANES)
                off16 = dvr[sl] - cbase
                valid = (off16 >= 0) & (off16 < C_ROWS)
                orr[slo] = jnp.where(valid, off16, C_ROWS)
                wrr[slo] = jnp.where(valid, vvr[sl], 0.0)
            for g in range(SB // LANES):
                w16 = wrr[pl.ds(g * LANES, LANES)]
                for j2 in range(LANES):
                    e = g * LANES + j2
                    bc = jnp.full((LANES,), w16[j2])
                    rre = rr.at[e]
                    for k in range(8):
                        slk = pl.ds(k * LANES, LANES)
                        rre[slk] = rre[slk] * bc
            pltpu.async_copy(rr, acc.at[orr], ssem, add=True)

            @pl.when(b >= 2)
            def _():
                pltpu.make_async_copy(rows.at[0], acc.at[off_r.at[0]],
                                      ssem).wait()

            @pl.when(b <= NSB - 3)
            def _():
                row2, hh2 = _sub(b + 2)
                pltpu.async_copy(x_hbm.at[src_v.at[row2, pl.ds(hh2, SB)]],
                                 rows.at[lax.rem(b + 2, 4)], gsem)
            return 0
        lax.fori_loop(0, NSB, _edge, 0)

        pltpu.make_async_copy(rows.at[0], acc.at[off_r.at[0]], ssem).wait()
        pltpu.make_async_copy(rows.at[0], acc.at[off_r.at[0]], ssem).wait()
        plsc.subcore_barrier()

        pltpu.sync_copy(
            acc.at[pl.ds(t * ROWS_PER_TILE, ROWS_PER_TILE)],
            out_hbm.at[pl.ds(cbase + t * ROWS_PER_TILE, ROWS_PER_TILE)])
        return 0
    lax.fori_loop(0, N_CHUNKS // 2, _pass, 0)


def _sc_agg(x, src_r, dst_r, val_r):
    mesh = plsc.VectorSubcoreMesh(core_axis_name="c", subcore_axis_name="s",
                                  num_cores=N_CORES, num_subcores=N_SUBCORES)
    f = pl.kernel(
        _sc_agg_body,
        out_type=jax.ShapeDtypeStruct((N_PAD, D_FEAT), jnp.float32),
        mesh=mesh,
        scratch_types=[
            pltpu.VMEM((NB, EB), jnp.int32),
            pltpu.VMEM((NB, EB), jnp.int32),
            pltpu.VMEM((NB, EB), jnp.float32),
            pltpu.VMEM((4, SB, D_FEAT), jnp.float32),
            pltpu.VMEM((4, SB), jnp.int32),
            pltpu.VMEM((4, SB), jnp.float32),
            pltpu.VMEM((8, EB), jnp.float32),
            pltpu.VMEM((8, EB), jnp.float32),
            pltpu.VMEM((3136,), jnp.float32),
            pltpu.VMEM((8, D_FEAT), jnp.float32),
            pltpu.VMEM_SHARED((16 * 3136,), jnp.float32),
            pltpu.VMEM_SHARED((16 * 3136,), jnp.float32),
            pltpu.VMEM_SHARED((ACC_ROWS, D_FEAT), jnp.float32),
            pltpu.SemaphoreType.DMA,
            pltpu.SemaphoreType.DMA,
            pltpu.SemaphoreType.DMA,
            pltpu.SemaphoreType.DMA,
        ],
    )
    return f(x, src_r, dst_r, val_r)


def _prep_edges(edge_index, val):
    e = val.shape[0]
    pad = E_PAD - e
    src = jnp.pad(edge_index[0], (0, pad)).reshape(E_PAD // EB, EB)
    dst = jnp.pad(edge_index[1], (0, pad)).reshape(E_PAD // EB, EB)
    v = jnp.pad(val, (0, pad)).reshape(E_PAD // EB, EB)
    return src, dst, v


def _dense_stage_body(a1_ref, a2_ref, w1_ref, w2_ref, b1_ref, b2_ref, o_ref):
    h1 = jnp.dot(a1_ref[...], w1_ref[...], preferred_element_type=jnp.float32)
    h2 = jnp.dot(a2_ref[...], w2_ref[...], preferred_element_type=jnp.float32)
    r1 = jnp.maximum(h1 + b1_ref[...], 0.0)
    r2 = jnp.maximum(h2 + b2_ref[...], 0.0)
    o_ref[...] = (r1 + r2) * 0.5


def _dense_stage(a1, a2, W1, W2, b1, b2):
    grid = (N_NODES // ROW_BLK,)
    blk = lambda i: (i, 0)
    full = lambda i: (0, 0)
    return pl.pallas_call(
        _dense_stage_body,
        grid=grid,
        in_specs=[
            pl.BlockSpec((ROW_BLK, D_FEAT), blk),
            pl.BlockSpec((ROW_BLK, D_FEAT), blk),
            pl.BlockSpec((D_FEAT, D_FEAT), full),
            pl.BlockSpec((D_FEAT, D_FEAT), full),
            pl.BlockSpec((1, D_FEAT), full),
            pl.BlockSpec((1, D_FEAT), full),
        ],
        out_specs=pl.BlockSpec((ROW_BLK, D_FEAT), blk),
        out_shape=jax.ShapeDtypeStruct((N_NODES, D_FEAT), jnp.float32),
    )(a1, a2, W1, W2, b1.reshape(1, -1), b2.reshape(1, -1))


def kernel(com_emb, pos_emb, demand_edge_index, supply_edge_index,
           comflow_edge_index, posflow_edge_index,
           demand_val, supply_val, comflow_val, posflow_val,
           W_demand, b_demand, W_supply, b_supply,
           W_comflow, b_comflow, W_posflow, b_posflow):
    d_src, d_dst, d_val = _prep_edges(demand_edge_index, demand_val)
    s_src, s_dst, s_val = _prep_edges(supply_edge_index, supply_val)
    c_src, c_dst, c_val = _prep_edges(comflow_edge_index, comflow_val)
    p_src, p_dst, p_val = _prep_edges(posflow_edge_index, posflow_val)

    agg_d = _sc_agg(com_emb, d_src, d_dst, d_val)
    agg_s = _sc_agg(pos_emb, s_src, s_dst, s_val)
    agg_c = _sc_agg(com_emb, c_src, c_dst, c_val)
    agg_p = _sc_agg(pos_emb, p_src, p_dst, p_val)

    com_out = _dense_stage(agg_s, agg_c, W_supply, W_comflow, b_supply, b_comflow)
    pos_out = _dense_stage(agg_d, agg_p, W_demand, W_posflow, b_demand, b_posflow)
    return (com_out, pos_out)


# speedup vs baseline: 1.0832x; 1.0007x over previous
import functools

import jax
import jax.numpy as jnp
from jax import lax
from jax.experimental import pallas as pl
from jax.experimental.pallas import tpu as pltpu
from jax.experimental.pallas import tpu_sc as plsc

N_NODES = 50000
D_FEAT = 128
LANES = 16
N_SUBCORES = 16
N_CORES = 2

EB = 128
SB = 16
RING = 16
NB = 80
NSB = NB * (EB // SB)
ET = NB * EB
E_PAD = N_SUBCORES * ET
C_ROWS = 6272
N_CHUNKS = 8
N_PAD = N_CHUNKS * C_ROWS
ACC_ROWS = C_ROWS + 8
ROWS_PER_TILE = C_ROWS // N_SUBCORES

ROW_BLK = 1000


def _rsqrt_sc(z):
    yi = jnp.int32(0x5F3759DF) - (lax.bitcast_convert_type(z, jnp.int32) >> 1)
    y = lax.bitcast_convert_type(yi, jnp.float32)
    for _ in range(3):
        y = y * (1.5 - 0.5 * z * y * y)
    return y


def _sc_agg_body(x_hbm, src_hbm, dst_hbm, val_hbm, out_hbm,
                 src_v, dst_v, val_v, rows, off_r, wm_r, dsb, ddb,
                 zb1, zb2, deg_s, deg_d, acc, p1sem, p2sem, gsem, ssem):
    t = lax.axis_index("s")
    core = lax.axis_index("c")

    pltpu.sync_copy(src_hbm.at[pl.ds(t * NB, NB)], src_v)
    pltpu.sync_copy(dst_hbm.at[pl.ds(t * NB, NB)], dst_v)
    pltpu.sync_copy(val_hbm.at[pl.ds(t * NB, NB)], val_v)

    zero16 = jnp.zeros((LANES,), jnp.float32)

    def _z1(i, _):
        zb1[pl.ds(i * LANES, LANES)] = zero16
        return 0
    lax.fori_loop(0, 3136 // LANES, _z1, 0)

    def _z2(i, _):
        for k in range(8):
            zb2[i, pl.ds(k * LANES, LANES)] = zero16
        return 0
    lax.fori_loop(0, 8, _z2, 0)

    pltpu.sync_copy(zb1, deg_s.at[pl.ds(t * 3136, 3136)])
    pltpu.sync_copy(zb1, deg_d.at[pl.ds(t * 3136, 3136)])
    plsc.subcore_barrier()

    W1 = 8
    for b in range(W1):
        pltpu.async_copy(val_v.at[b], deg_s.at[src_v.at[b]], p1sem, add=True)
        pltpu.async_copy(val_v.at[b], deg_d.at[dst_v.at[b]], p1sem, add=True)

    def _p1(b, _):
        @pl.when(b <= NB - W1 - 1)
        def _():
            pltpu.async_copy(val_v.at[b + W1], deg_s.at[src_v.at[b + W1]],
                             p1sem, add=True)
            pltpu.async_copy(val_v.at[b + W1], deg_d.at[dst_v.at[b + W1]],
                             p1sem, add=True)
        pltpu.make_async_copy(val_v.at[b], deg_s.at[src_v.at[b]], p1sem).wait()
        pltpu.make_async_copy(val_v.at[b], deg_d.at[dst_v.at[b]], p1sem).wait()
        return 0
    lax.fori_loop(0, NB, _p1, 0)
    plsc.subcore_barrier()

    W2 = 6
    for b in range(W2):
        pltpu.async_copy(deg_s.at[src_v.at[b]], dsb.at[b % 8], p2sem)
        pltpu.async_copy(deg_d.at[dst_v.at[b]], ddb.at[b % 8], p2sem)

    def _p2(b, _):
        @pl.when(b <= NB - W2 - 1)
        def _():
            s = lax.rem(b + W2, 8)
            pltpu.async_copy(deg_s.at[src_v.at[b + W2]], dsb.at[s], p2sem)
            pltpu.async_copy(deg_d.at[dst_v.at[b + W2]], ddb.at[s], p2sem)
        s = lax.rem(b, 8)
        pltpu.make_async_copy(deg_s.at[src_v.at[b]], dsb.at[s], p2sem).wait()
        pltpu.make_async_copy(deg_d.at[dst_v.at[b]], ddb.at[s], p2sem).wait()
        dsr = dsb.at[s]
        ddr = ddb.at[s]
        vvr = val_v.at[b]
        for g in range(EB // LANES):
            sl = pl.ds(g * LANES, LANES)
            z = (jnp.maximum(dsr[sl], 1e-12) * jnp.maximum(ddr[sl], 1e-12))
            vvr[sl] = vvr[sl] * _rsqrt_sc(z)
        return 0
    lax.fori_loop(0, NB, _p2, 0)

    def _sub(b):
        return lax.div(b, EB // SB), lax.rem(b, EB // SB) * SB

    def _pass(p, _):
        cbase = (core * (N_CHUNKS // 2) + p) * C_ROWS

        def _za(i, _):
            pltpu.sync_copy(zb2, acc.at[pl.ds(t * ROWS_PER_TILE + i * 8, 8)])
            return 0
        lax.fori_loop(0, ROWS_PER_TILE // 8, _za, 0)
        plsc.subcore_barrier()


        for b0 in range(RING - 2):
            r0, h0 = b0 // (EB // SB), (b0 % (EB // SB)) * SB
            pltpu.async_copy(x_hbm.at[src_v.at[r0, pl.ds(h0, SB)]],
                             rows.at[b0], gsem)

        def _edge(b, _):
            slot = lax.rem(b, RING)
            row, hh = _sub(b)
            pltpu.make_async_copy(x_hbm.at[src_v.at[row, pl.ds(hh, SB)]],
                                  rows.at[slot], gsem).wait()
            rr = rows.at[slot]
            orr = off_r.at[slot]
            wrr = wm_r.at[slot]
            sl = pl.ds(hh, LANES)
            off16 = dst_v.at[row][sl] - cbase
            valid = (off16 >= 0) & (off16 < C_ROWS)
            orr[pl.ds(0, LANES)] = jnp.where(valid, off16, C_ROWS)
            w16 = jnp.where(valid, val_v.at[row][sl], 0.0)
            wrr[pl.ds(0, LANES)] = w16
            for j2 in range(LANES):
                bc = jnp.full((LANES,), w16[j2])
                rre = rr.at[j2]
                for k in range(8):
                    slk = pl.ds(k * LANES, LANES)
                    rre[slk] = rre[slk] * bc
            pltpu.async_copy(rr, acc.at[orr], ssem, add=True)

            @pl.when(b >= 2)
            def _():
                pltpu.make_async_copy(rows.at[0], acc.at[off_r.at[0]],
                                      ssem).wait()

            @pl.when(b <= NSB - RING + 1)
            def _():
                row2, hh2 = _sub(b + RING - 2)
                pltpu.async_copy(x_hbm.at[src_v.at[row2, pl.ds(hh2, SB)]],
                                 rows.at[lax.rem(b + RING - 2, RING)], gsem)
            return 0
        lax.fori_loop(0, NSB, _edge, 0)

        pltpu.make_async_copy(rows.at[0], acc.at[off_r.at[0]], ssem).wait()
        pltpu.make_async_copy(rows.at[0], acc.at[off_r.at[0]], ssem).wait()
        plsc.subcore_barrier()

        pltpu.sync_copy(
            acc.at[pl.ds(t * ROWS_PER_TILE, ROWS_PER_TILE)],
            out_hbm.at[pl.ds(cbase + t * ROWS_PER_TILE, ROWS_PER_TILE)])
        return 0
    lax.fori_loop(0, N_CHUNKS // 2, _pass, 0)


def _sc_agg(x, src_r, dst_r, val_r):
    mesh = plsc.VectorSubcoreMesh(core_axis_name="c", subcore_axis_name="s",
                                  num_cores=N_CORES, num_subcores=N_SUBCORES)
    f = pl.kernel(
        _sc_agg_body,
        out_type=jax.ShapeDtypeStruct((N_PAD, D_FEAT), jnp.float32),
        mesh=mesh,
        scratch_types=[
            pltpu.VMEM((NB, EB), jnp.int32),
            pltpu.VMEM((NB, EB), jnp.int32),
            pltpu.VMEM((NB, EB), jnp.float32),
            pltpu.VMEM((RING, SB, D_FEAT), jnp.float32),
            pltpu.VMEM((RING, SB), jnp.int32),
            pltpu.VMEM((RING, SB), jnp.float32),
            pltpu.VMEM((8, EB), jnp.float32),
            pltpu.VMEM((8, EB), jnp.float32),
            pltpu.VMEM((3136,), jnp.float32),
            pltpu.VMEM((8, D_FEAT), jnp.float32),
            pltpu.VMEM_SHARED((16 * 3136,), jnp.float32),
            pltpu.VMEM_SHARED((16 * 3136,), jnp.float32),
            pltpu.VMEM_SHARED((ACC_ROWS, D_FEAT), jnp.float32),
            pltpu.SemaphoreType.DMA,
            pltpu.SemaphoreType.DMA,
            pltpu.SemaphoreType.DMA,
            pltpu.SemaphoreType.DMA,
        ],
    )
    return f(x, src_r, dst_r, val_r)


def _prep_edges(edge_index, val):
    e = val.shape[0]
    pad = E_PAD - e
    src = jnp.pad(edge_index[0], (0, pad)).reshape(E_PAD // EB, EB)
    dst = jnp.pad(edge_index[1], (0, pad)).reshape(E_PAD // EB, EB)
    v = jnp.pad(val, (0, pad)).reshape(E_PAD // EB, EB)
    return src, dst, v


def _dense_stage_body(a1_ref, a2_ref, w1_ref, w2_ref, b1_ref, b2_ref, o_ref):
    h1 = jnp.dot(a1_ref[...], w1_ref[...], preferred_element_type=jnp.float32)
    h2 = jnp.dot(a2_ref[...], w2_ref[...], preferred_element_type=jnp.float32)
    r1 = jnp.maximum(h1 + b1_ref[...], 0.0)
    r2 = jnp.maximum(h2 + b2_ref[...], 0.0)
    o_ref[...] = (r1 + r2) * 0.5


def _dense_stage(a1, a2, W1, W2, b1, b2):
    grid = (N_NODES // ROW_BLK,)
    blk = lambda i: (i, 0)
    full = lambda i: (0, 0)
    return pl.pallas_call(
        _dense_stage_body,
        grid=grid,
        in_specs=[
            pl.BlockSpec((ROW_BLK, D_FEAT), blk),
            pl.BlockSpec((ROW_BLK, D_FEAT), blk),
            pl.BlockSpec((D_FEAT, D_FEAT), full),
            pl.BlockSpec((D_FEAT, D_FEAT), full),
            pl.BlockSpec((1, D_FEAT), full),
            pl.BlockSpec((1, D_FEAT), full),
        ],
        out_specs=pl.BlockSpec((ROW_BLK, D_FEAT), blk),
        out_shape=jax.ShapeDtypeStruct((N_NODES, D_FEAT), jnp.float32),
    )(a1, a2, W1, W2, b1.reshape(1, -1), b2.reshape(1, -1))


def kernel(com_emb, pos_emb, demand_edge_index, supply_edge_index,
           comflow_edge_index, posflow_edge_index,
           demand_val, supply_val, comflow_val, posflow_val,
           W_demand, b_demand, W_supply, b_supply,
           W_comflow, b_comflow, W_posflow, b_posflow):
    d_src, d_dst, d_val = _prep_edges(demand_edge_index, demand_val)
    s_src, s_dst, s_val = _prep_edges(supply_edge_index, supply_val)
    c_src, c_dst, c_val = _prep_edges(comflow_edge_index, comflow_val)
    p_src, p_dst, p_val = _prep_edges(posflow_edge_index, posflow_val)

    agg_d = _sc_agg(com_emb, d_src, d_dst, d_val)
    agg_s = _sc_agg(pos_emb, s_src, s_dst, s_val)
    agg_c = _sc_agg(com_emb, c_src, c_dst, c_val)
    agg_p = _sc_agg(pos_emb, p_src, p_dst, p_val)

    com_out = _dense_stage(agg_s, agg_c, W_supply, W_comflow, b_supply, b_comflow)
    pos_out = _dense_stage(agg_d, agg_p, W_demand, W_posflow, b_demand, b_posflow)
    return (com_out, pos_out)


# speedup vs baseline: 1.9954x; 1.8421x over previous
import functools

import numpy as np

import jax
import jax.numpy as jnp
from jax import lax
from jax.experimental import pallas as pl
from jax.experimental.pallas import tpu as pltpu
from jax.experimental.pallas import tpu_sc as plsc

N_NODES = 50000
D_FEAT = 128
LANES = 16
N_SUBCORES = 16
N_CORES = 2

EB = 128
SB = 16
RING = 16
SRING = 4
NB = 80
NSB = NB * (EB // SB)
ET = NB * EB
E_PAD = N_SUBCORES * ET
C_ROWS = 6272
N_CHUNKS = 8
N_PAD = N_CHUNKS * C_ROWS
ACC_ROWS = C_ROWS + 8
ROWS_PER_TILE = C_ROWS // N_SUBCORES

ROW_BLK = 1000

_COL_PERM = np.empty((D_FEAT,), np.int32)
for _k in range(4):
    for _j in range(16):
        _COL_PERM[32 * _k + _j] = 2 * (16 * _k + _j)
        _COL_PERM[32 * _k + 16 + _j] = 2 * (16 * _k + _j) + 1


def _rsqrt_sc(z):
    yi = jnp.int32(0x5F3759DF) - (lax.bitcast_convert_type(z, jnp.int32) >> 1)
    y = lax.bitcast_convert_type(yi, jnp.float32)
    for _ in range(3):
        y = y * (1.5 - 0.5 * z * y * y)
    return y


def _sc_agg_body(x_hbm, src_hbm, dst_hbm, val_hbm, out_hbm,
                 src_v, dst_v, val_v, rows, stg, off_r, wm_r, dsb, ddb,
                 zb1, zb2, deg_s, deg_d, acc, p1sem, p2sem, gsem, ssem):
    t = lax.axis_index("s")
    core = lax.axis_index("c")

    pltpu.sync_copy(src_hbm.at[pl.ds(t * NB, NB)], src_v)
    pltpu.sync_copy(dst_hbm.at[pl.ds(t * NB, NB)], dst_v)
    pltpu.sync_copy(val_hbm.at[pl.ds(t * NB, NB)], val_v)

    zero16 = jnp.zeros((LANES,), jnp.float32)

    def _z1(i, _):
        zb1[pl.ds(i * LANES, LANES)] = zero16
        return 0
    lax.fori_loop(0, 3136 // LANES, _z1, 0)

    def _z2(i, _):
        for k in range(8):
            zb2[i, pl.ds(k * LANES, LANES)] = zero16
        return 0
    lax.fori_loop(0, 8, _z2, 0)

    pltpu.sync_copy(zb1, deg_s.at[pl.ds(t * 3136, 3136)])
    pltpu.sync_copy(zb1, deg_d.at[pl.ds(t * 3136, 3136)])
    plsc.subcore_barrier()

    W1 = 8
    for b in range(W1):
        pltpu.async_copy(val_v.at[b], deg_s.at[src_v.at[b]], p1sem, add=True)
        pltpu.async_copy(val_v.at[b], deg_d.at[dst_v.at[b]], p1sem, add=True)

    def _p1(b, _):
        @pl.when(b <= NB - W1 - 1)
        def _():
            pltpu.async_copy(val_v.at[b + W1], deg_s.at[src_v.at[b + W1]],
                             p1sem, add=True)
            pltpu.async_copy(val_v.at[b + W1], deg_d.at[dst_v.at[b + W1]],
                             p1sem, add=True)
        pltpu.make_async_copy(val_v.at[b], deg_s.at[src_v.at[b]], p1sem).wait()
        pltpu.make_async_copy(val_v.at[b], deg_d.at[dst_v.at[b]], p1sem).wait()
        return 0
    lax.fori_loop(0, NB, _p1, 0)
    plsc.subcore_barrier()

    W2 = 6
    for b in range(W2):
        pltpu.async_copy(deg_s.at[src_v.at[b]], dsb.at[b % 8], p2sem)
        pltpu.async_copy(deg_d.at[dst_v.at[b]], ddb.at[b % 8], p2sem)

    def _p2(b, _):
        @pl.when(b <= NB - W2 - 1)
        def _():
            s = lax.rem(b + W2, 8)
            pltpu.async_copy(deg_s.at[src_v.at[b + W2]], dsb.at[s], p2sem)
            pltpu.async_copy(deg_d.at[dst_v.at[b + W2]], ddb.at[s], p2sem)
        s = lax.rem(b, 8)
        pltpu.make_async_copy(deg_s.at[src_v.at[b]], dsb.at[s], p2sem).wait()
        pltpu.make_async_copy(deg_d.at[dst_v.at[b]], ddb.at[s], p2sem).wait()
        dsr = dsb.at[s]
        ddr = ddb.at[s]
        vvr = val_v.at[b]
        for g in range(EB // LANES):
            sl = pl.ds(g * LANES, LANES)
            z = (jnp.maximum(dsr[sl], 1e-12) * jnp.maximum(ddr[sl], 1e-12))
            vvr[sl] = vvr[sl] * _rsqrt_sc(z)
        return 0
    lax.fori_loop(0, NB, _p2, 0)

    def _sub(b):
        return lax.div(b, EB // SB), lax.rem(b, EB // SB) * SB

    def _pass(p, _):
        cbase = (core * (N_CHUNKS // 2) + p) * C_ROWS

        def _za(i, _):
            pltpu.sync_copy(zb2, acc.at[pl.ds(t * ROWS_PER_TILE + i * 8, 8)])
            return 0
        lax.fori_loop(0, ROWS_PER_TILE // 8, _za, 0)
        plsc.subcore_barrier()

        for b0 in range(RING - 2):
            r0, h0 = b0 // (EB // SB), (b0 % (EB // SB)) * SB
            pltpu.async_copy(x_hbm.at[src_v.at[r0, pl.ds(h0, SB)]],
                             rows.at[b0], gsem)

        def _edge(b, _):
            slot = lax.rem(b, RING)
            sslot = lax.rem(b, SRING)
            row, hh = _sub(b)
            pltpu.make_async_copy(x_hbm.at[src_v.at[row, pl.ds(hh, SB)]],
                                  rows.at[slot], gsem).wait()

            @pl.when(b >= SRING)
            def _():
                pltpu.make_async_copy(stg.at[0], acc.at[off_r.at[0]],
                                      ssem).wait()

            orr = off_r.at[slot]
            wrr = wm_r.at[slot]
            sl = pl.ds(hh, LANES)
            off16 = dst_v.at[row][sl] - cbase
            valid = (off16 >= 0) & (off16 < C_ROWS)
            orr[pl.ds(0, LANES)] = jnp.where(valid, off16, C_ROWS)
            w16 = jnp.where(valid, val_v.at[row][sl], 0.0)
            wrr[pl.ds(0, LANES)] = w16
            rr = rows.at[slot]
            sr = stg.at[sslot]
            for j2 in range(LANES):
                bc = jnp.full((LANES,), w16[j2])
                rre = rr.at[j2]
                ste = sr.at[j2]
                for k in range(4):
                    v = rre[pl.ds(k * LANES, LANES)]
                    f0 = lax.bitcast_convert_type(v << 16, jnp.float32)
                    f1 = lax.bitcast_convert_type(
                        v & jnp.int32(-65536), jnp.float32)
                    ste[pl.ds(32 * k, LANES)] = f0 * bc
                    ste[pl.ds(32 * k + LANES, LANES)] = f1 * bc
            pltpu.async_copy(sr, acc.at[orr], ssem, add=True)

            @pl.when(b <= NSB - RING + 1)
            def _():
                row2, hh2 = _sub(b + RING - 2)
                pltpu.async_copy(x_hbm.at[src_v.at[row2, pl.ds(hh2, SB)]],
                                 rows.at[lax.rem(b + RING - 2, RING)], gsem)
            return 0
        lax.fori_loop(0, NSB, _edge, 0)

        for _ in range(SRING):
            pltpu.make_async_copy(stg.at[0], acc.at[off_r.at[0]], ssem).wait()
        plsc.subcore_barrier()

        pltpu.sync_copy(
            acc.at[pl.ds(t * ROWS_PER_TILE, ROWS_PER_TILE)],
            out_hbm.at[pl.ds(cbase + t * ROWS_PER_TILE, ROWS_PER_TILE)])
        return 0
    lax.fori_loop(0, N_CHUNKS // 2, _pass, 0)


def _sc_agg(x_packed, src_r, dst_r, val_r):
    mesh = plsc.VectorSubcoreMesh(core_axis_name="c", subcore_axis_name="s",
                                  num_cores=N_CORES, num_subcores=N_SUBCORES)
    f = pl.kernel(
        _sc_agg_body,
        out_type=jax.ShapeDtypeStruct((N_PAD, D_FEAT), jnp.float32),
        mesh=mesh,
        compiler_params=pltpu.CompilerParams(use_tc_tiling_on_sc=False),
        scratch_types=[
            pltpu.VMEM((NB, EB), jnp.int32),
            pltpu.VMEM((NB, EB), jnp.int32),
            pltpu.VMEM((NB, EB), jnp.float32),
            pltpu.VMEM((RING, SB, D_FEAT // 2), jnp.int32),
            pltpu.VMEM((SRING, SB, D_FEAT), jnp.float32),
            pltpu.VMEM((RING, SB), jnp.int32),
            pltpu.VMEM((RING, SB), jnp.float32),
            pltpu.VMEM((8, EB), jnp.float32),
            pltpu.VMEM((8, EB), jnp.float32),
            pltpu.VMEM((3136,), jnp.float32),
            pltpu.VMEM((8, D_FEAT), jnp.float32),
            pltpu.VMEM_SHARED((16 * 3136,), jnp.float32),
            pltpu.VMEM_SHARED((16 * 3136,), jnp.float32),
            pltpu.VMEM_SHARED((ACC_ROWS, D_FEAT), jnp.float32),
            pltpu.SemaphoreType.DMA,
            pltpu.SemaphoreType.DMA,
            pltpu.SemaphoreType.DMA,
            pltpu.SemaphoreType.DMA,
        ],
    )
    return f(x_packed, src_r, dst_r, val_r)


def _prep_edges(edge_index, val):
    e = val.shape[0]
    pad = E_PAD - e
    src = jnp.pad(edge_index[0], (0, pad)).reshape(E_PAD // EB, EB)
    dst = jnp.pad(edge_index[1], (0, pad)).reshape(E_PAD // EB, EB)
    v = jnp.pad(val, (0, pad)).reshape(E_PAD // EB, EB)
    return src, dst, v


def _pack_bf16(x):
    xb = x.astype(jnp.bfloat16).reshape(x.shape[0], D_FEAT // 2, 2)
    return lax.bitcast_convert_type(xb, jnp.int32)


def _dense_stage_body(a1_ref, a2_ref, w1_ref, w2_ref, b1_ref, b2_ref, o_ref):
    h1 = jnp.dot(a1_ref[...], w1_ref[...], preferred_element_type=jnp.float32)
    h2 = jnp.dot(a2_ref[...], w2_ref[...], preferred_element_type=jnp.float32)
    r1 = jnp.maximum(h1 + b1_ref[...], 0.0)
    r2 = jnp.maximum(h2 + b2_ref[...], 0.0)
    o_ref[...] = (r1 + r2) * 0.5


def _dense_stage(a1, a2, W1, W2, b1, b2):
    grid = (N_NODES // ROW_BLK,)
    blk = lambda i: (i, 0)
    full = lambda i: (0, 0)
    return pl.pallas_call(
        _dense_stage_body,
        grid=grid,
        in_specs=[
            pl.BlockSpec((ROW_BLK, D_FEAT), blk),
            pl.BlockSpec((ROW_BLK, D_FEAT), blk),
            pl.BlockSpec((D_FEAT, D_FEAT), full),
            pl.BlockSpec((D_FEAT, D_FEAT), full),
            pl.BlockSpec((1, D_FEAT), full),
            pl.BlockSpec((1, D_FEAT), full),
        ],
        out_specs=pl.BlockSpec((ROW_BLK, D_FEAT), blk),
        out_shape=jax.ShapeDtypeStruct((N_NODES, D_FEAT), jnp.float32),
    )(a1, a2, W1[_COL_PERM, :], W2[_COL_PERM, :],
      b1.reshape(1, -1), b2.reshape(1, -1))


def kernel(com_emb, pos_emb, demand_edge_index, supply_edge_index,
           comflow_edge_index, posflow_edge_index,
           demand_val, supply_val, comflow_val, posflow_val,
           W_demand, b_demand, W_supply, b_supply,
           W_comflow, b_comflow, W_posflow, b_posflow):
    d_src, d_dst, d_val = _prep_edges(demand_edge_index, demand_val)
    s_src, s_dst, s_val = _prep_edges(supply_edge_index, supply_val)
    c_src, c_dst, c_val = _prep_edges(comflow_edge_index, comflow_val)
    p_src, p_dst, p_val = _prep_edges(posflow_edge_index, posflow_val)

    com16 = _pack_bf16(com_emb)
    pos16 = _pack_bf16(pos_emb)

    agg_d = _sc_agg(com16, d_src, d_dst, d_val)
    agg_s = _sc_agg(pos16, s_src, s_dst, s_val)
    agg_c = _sc_agg(com16, c_src, c_dst, c_val)
    agg_p = _sc_agg(pos16, p_src, p_dst, p_val)

    com_out = _dense_stage(agg_s, agg_c, W_supply, W_comflow, b_supply, b_comflow)
    pos_out = _dense_stage(agg_d, agg_p, W_demand, W_posflow, b_demand, b_posflow)
    return (com_out, pos_out)


# speedup vs baseline: 2.5130x; 1.2594x over previous
import functools

import numpy as np

import jax
import jax.numpy as jnp
from jax import lax
from jax.experimental import pallas as pl
from jax.experimental.pallas import tpu as pltpu
from jax.experimental.pallas import tpu_sc as plsc

N_NODES = 50000
D_FEAT = 128
LANES = 16
N_SUBCORES = 16
N_CORES = 2

EB = 128
SB = 16
RING = 4
SRING = 4
NB = 80
NSB = NB * (EB // SB)
ET = NB * EB
E_PAD = N_SUBCORES * ET
C_ROWS = 8448
N_CHUNKS = 6
N_PAD = N_CHUNKS * C_ROWS
ACC_ROWS = C_ROWS + 8
ROWS_PER_TILE = C_ROWS // N_SUBCORES

ROW_BLK = 1000

_COL_PERM = np.empty((D_FEAT,), np.int32)
for _k in range(4):
    for _j in range(16):
        _COL_PERM[32 * _k + _j] = 2 * (16 * _k + _j)
        _COL_PERM[32 * _k + 16 + _j] = 2 * (16 * _k + _j) + 1


def _rsqrt_sc(z):
    yi = jnp.int32(0x5F3759DF) - (lax.bitcast_convert_type(z, jnp.int32) >> 1)
    y = lax.bitcast_convert_type(yi, jnp.float32)
    for _ in range(3):
        y = y * (1.5 - 0.5 * z * y * y)
    return y


def _sc_agg_body(x_hbm, src_hbm, dst_hbm, val_hbm, out_hbm,
                 src_v, dst_v, val_v, rows, stg, off_r, wm_r, dsb, ddb,
                 zb1, zb2, deg_s, deg_d, acc, p1sem, p2sem, gsem, ssem):
    t = lax.axis_index("s")
    core = lax.axis_index("c")

    pltpu.sync_copy(src_hbm.at[pl.ds(t * NB, NB)], src_v)
    pltpu.sync_copy(dst_hbm.at[pl.ds(t * NB, NB)], dst_v)
    pltpu.sync_copy(val_hbm.at[pl.ds(t * NB, NB)], val_v)

    zero16 = jnp.zeros((LANES,), jnp.float32)

    def _z1(i, _):
        zb1[pl.ds(i * LANES, LANES)] = zero16
        return 0
    lax.fori_loop(0, 3136 // LANES, _z1, 0)

    def _z2(i, _):
        for k in range(8):
            zb2[i, pl.ds(k * LANES, LANES)] = zero16
        return 0
    lax.fori_loop(0, 8, _z2, 0)

    pltpu.sync_copy(zb1, deg_s.at[pl.ds(t * 3136, 3136)])
    pltpu.sync_copy(zb1, deg_d.at[pl.ds(t * 3136, 3136)])
    plsc.subcore_barrier()

    W1 = 8
    for b in range(W1):
        pltpu.async_copy(val_v.at[b], deg_s.at[src_v.at[b]], p1sem, add=True)
        pltpu.async_copy(val_v.at[b], deg_d.at[dst_v.at[b]], p1sem, add=True)

    def _p1(b, _):
        @pl.when(b <= NB - W1 - 1)
        def _():
            pltpu.async_copy(val_v.at[b + W1], deg_s.at[src_v.at[b + W1]],
                             p1sem, add=True)
            pltpu.async_copy(val_v.at[b + W1], deg_d.at[dst_v.at[b + W1]],
                             p1sem, add=True)
        pltpu.make_async_copy(val_v.at[b], deg_s.at[src_v.at[b]], p1sem).wait()
        pltpu.make_async_copy(val_v.at[b], deg_d.at[dst_v.at[b]], p1sem).wait()
        return 0
    lax.fori_loop(0, NB, _p1, 0)
    plsc.subcore_barrier()

    W2 = 6
    for b in range(W2):
        pltpu.async_copy(deg_s.at[src_v.at[b]], dsb.at[b % 8], p2sem)
        pltpu.async_copy(deg_d.at[dst_v.at[b]], ddb.at[b % 8], p2sem)

    def _p2(b, _):
        @pl.when(b <= NB - W2 - 1)
        def _():
            s = lax.rem(b + W2, 8)
            pltpu.async_copy(deg_s.at[src_v.at[b + W2]], dsb.at[s], p2sem)
            pltpu.async_copy(deg_d.at[dst_v.at[b + W2]], ddb.at[s], p2sem)
        s = lax.rem(b, 8)
        pltpu.make_async_copy(deg_s.at[src_v.at[b]], dsb.at[s], p2sem).wait()
        pltpu.make_async_copy(deg_d.at[dst_v.at[b]], ddb.at[s], p2sem).wait()
        dsr = dsb.at[s]
        ddr = ddb.at[s]
        vvr = val_v.at[b]
        for g in range(EB // LANES):
            sl = pl.ds(g * LANES, LANES)
            z = (jnp.maximum(dsr[sl], 1e-12) * jnp.maximum(ddr[sl], 1e-12))
            vvr[sl] = vvr[sl] * _rsqrt_sc(z)
        return 0
    lax.fori_loop(0, NB, _p2, 0)

    def _sub(b):
        return lax.div(b, EB // SB), lax.rem(b, EB // SB) * SB

    def _pass(p, _):
        cbase = (core * (N_CHUNKS // 2) + p) * C_ROWS

        def _za(i, _):
            pltpu.sync_copy(zb2, acc.at[pl.ds(t * ROWS_PER_TILE + i * 8, 8)])
            return 0
        lax.fori_loop(0, ROWS_PER_TILE // 8, _za, 0)
        plsc.subcore_barrier()

        for b0 in range(RING - 2):
            r0, h0 = b0 // (EB // SB), (b0 % (EB // SB)) * SB
            pltpu.async_copy(x_hbm.at[src_v.at[r0, pl.ds(h0, SB)]],
                             rows.at[b0], gsem)

        def _edge(b, _):
            slot = lax.rem(b, RING)
            sslot = lax.rem(b, SRING)
            row, hh = _sub(b)
            pltpu.make_async_copy(x_hbm.at[src_v.at[row, pl.ds(hh, SB)]],
                                  rows.at[slot], gsem).wait()

            @pl.when(b >= SRING)
            def _():
                pltpu.make_async_copy(stg.at[0], acc.at[off_r.at[0]],
                                      ssem).wait()

            orr = off_r.at[slot]
            wrr = wm_r.at[slot]
            sl = pl.ds(hh, LANES)
            off16 = dst_v.at[row][sl] - cbase
            valid = (off16 >= 0) & (off16 < C_ROWS)
            orr[pl.ds(0, LANES)] = jnp.where(valid, off16, C_ROWS)
            w16 = jnp.where(valid, val_v.at[row][sl], 0.0)
            wrr[pl.ds(0, LANES)] = w16
            rr = rows.at[slot]
            sr = stg.at[sslot]
            for j2 in range(LANES):
                bc = jnp.full((LANES,), w16[j2])
                rre = rr.at[j2]
                ste = sr.at[j2]
                for k in range(4):
                    v = rre[pl.ds(k * LANES, LANES)]
                    f0 = lax.bitcast_convert_type(v << 16, jnp.float32)
                    f1 = lax.bitcast_convert_type(
                        v & jnp.int32(-65536), jnp.float32)
                    ste[pl.ds(32 * k, LANES)] = f0 * bc
                    ste[pl.ds(32 * k + LANES, LANES)] = f1 * bc
            pltpu.async_copy(sr, acc.at[orr], ssem, add=True)

            @pl.when(b <= NSB - RING + 1)
            def _():
                row2, hh2 = _sub(b + RING - 2)
                pltpu.async_copy(x_hbm.at[src_v.at[row2, pl.ds(hh2, SB)]],
                                 rows.at[lax.rem(b + RING - 2, RING)], gsem)
            return 0
        lax.fori_loop(0, NSB, _edge, 0)

        for _ in range(SRING):
            pltpu.make_async_copy(stg.at[0], acc.at[off_r.at[0]], ssem).wait()
        plsc.subcore_barrier()

        pltpu.sync_copy(
            acc.at[pl.ds(t * ROWS_PER_TILE, ROWS_PER_TILE)],
            out_hbm.at[pl.ds(cbase + t * ROWS_PER_TILE, ROWS_PER_TILE)])
        return 0
    lax.fori_loop(0, N_CHUNKS // 2, _pass, 0)


def _sc_agg(x_packed, src_r, dst_r, val_r):
    mesh = plsc.VectorSubcoreMesh(core_axis_name="c", subcore_axis_name="s",
                                  num_cores=N_CORES, num_subcores=N_SUBCORES)
    f = pl.kernel(
        _sc_agg_body,
        out_type=jax.ShapeDtypeStruct((N_PAD, D_FEAT), jnp.float32),
        mesh=mesh,
        compiler_params=pltpu.CompilerParams(use_tc_tiling_on_sc=False),
        scratch_types=[
            pltpu.VMEM((NB, EB), jnp.int32),
            pltpu.VMEM((NB, EB), jnp.int32),
            pltpu.VMEM((NB, EB), jnp.float32),
            pltpu.VMEM((RING, SB, D_FEAT // 2), jnp.int32),
            pltpu.VMEM((SRING, SB, D_FEAT), jnp.float32),
            pltpu.VMEM((RING, SB), jnp.int32),
            pltpu.VMEM((RING, SB), jnp.float32),
            pltpu.VMEM((8, EB), jnp.float32),
            pltpu.VMEM((8, EB), jnp.float32),
            pltpu.VMEM((3136,), jnp.float32),
            pltpu.VMEM((8, D_FEAT), jnp.float32),
            pltpu.VMEM_SHARED((16 * 3136,), jnp.float32),
            pltpu.VMEM_SHARED((16 * 3136,), jnp.float32),
            pltpu.VMEM_SHARED((ACC_ROWS, D_FEAT), jnp.float32),
            pltpu.SemaphoreType.DMA,
            pltpu.SemaphoreType.DMA,
            pltpu.SemaphoreType.DMA,
            pltpu.SemaphoreType.DMA,
        ],
    )
    return f(x_packed, src_r, dst_r, val_r)


def _prep_edges(edge_index, val):
    e = val.shape[0]
    pad = E_PAD - e
    src = jnp.pad(edge_index[0], (0, pad)).reshape(E_PAD // EB, EB)
    dst = jnp.pad(edge_index[1], (0, pad)).reshape(E_PAD // EB, EB)
    v = jnp.pad(val, (0, pad)).reshape(E_PAD // EB, EB)
    return src, dst, v


def _pack_bf16(x):
    xb = x.astype(jnp.bfloat16).reshape(x.shape[0], D_FEAT // 2, 2)
    return lax.bitcast_convert_type(xb, jnp.int32)


def _dense_stage_body(a1_ref, a2_ref, w1_ref, w2_ref, b1_ref, b2_ref, o_ref):
    h1 = jnp.dot(a1_ref[...], w1_ref[...], preferred_element_type=jnp.float32)
    h2 = jnp.dot(a2_ref[...], w2_ref[...], preferred_element_type=jnp.float32)
    r1 = jnp.maximum(h1 + b1_ref[...], 0.0)
    r2 = jnp.maximum(h2 + b2_ref[...], 0.0)
    o_ref[...] = (r1 + r2) * 0.5


def _dense_stage(a1, a2, W1, W2, b1, b2):
    grid = (N_NODES // ROW_BLK,)
    blk = lambda i: (i, 0)
    full = lambda i: (0, 0)
    return pl.pallas_call(
        _dense_stage_body,
        grid=grid,
        in_specs=[
            pl.BlockSpec((ROW_BLK, D_FEAT), blk),
            pl.BlockSpec((ROW_BLK, D_FEAT), blk),
            pl.BlockSpec((D_FEAT, D_FEAT), full),
            pl.BlockSpec((D_FEAT, D_FEAT), full),
            pl.BlockSpec((1, D_FEAT), full),
            pl.BlockSpec((1, D_FEAT), full),
        ],
        out_specs=pl.BlockSpec((ROW_BLK, D_FEAT), blk),
        out_shape=jax.ShapeDtypeStruct((N_NODES, D_FEAT), jnp.float32),
    )(a1, a2, W1[_COL_PERM, :], W2[_COL_PERM, :],
      b1.reshape(1, -1), b2.reshape(1, -1))


def kernel(com_emb, pos_emb, demand_edge_index, supply_edge_index,
           comflow_edge_index, posflow_edge_index,
           demand_val, supply_val, comflow_val, posflow_val,
           W_demand, b_demand, W_supply, b_supply,
           W_comflow, b_comflow, W_posflow, b_posflow):
    d_src, d_dst, d_val = _prep_edges(demand_edge_index, demand_val)
    s_src, s_dst, s_val = _prep_edges(supply_edge_index, supply_val)
    c_src, c_dst, c_val = _prep_edges(comflow_edge_index, comflow_val)
    p_src, p_dst, p_val = _prep_edges(posflow_edge_index, posflow_val)

    com16 = _pack_bf16(com_emb)
    pos16 = _pack_bf16(pos_emb)

    agg_d = _sc_agg(com16, d_src, d_dst, d_val)
    agg_s = _sc_agg(pos16, s_src, s_dst, s_val)
    agg_c = _sc_agg(com16, c_src, c_dst, c_val)
    agg_p = _sc_agg(pos16, p_src, p_dst, p_val)

    com_out = _dense_stage(agg_s, agg_c, W_supply, W_comflow, b_supply, b_comflow)
    pos_out = _dense_stage(agg_d, agg_p, W_demand, W_posflow, b_demand, b_posflow)
    return (com_out, pos_out)
